# baseline (speedup 1.0000x reference)
import jax
import jax.numpy as jnp
from jax import lax
from jax.experimental import pallas as pl
from jax.experimental.pallas import tpu as pltpu
from jax.experimental.pallas import tpu_sc as plsc

N_NODES = 10000
N_EDGES = 320000
D = 128
DH = D // 2

NC = 2
NS = 16
NPAD = 10112
ROWS_PER_TILE = NPAD // NS
K = 128
E_PAD = 327680
EDGES_PER_TILE = E_PAD // NS
CHUNKS_PER_TILE = EDGES_PER_TILE // K
CW = 16
RING = 4

_CHUNK_SIZES = (128, 128, 128, 128, 120)

BLK = NPAD // 4


def _make_sc_agg(with_cnt: bool):
    out_type = [jax.ShapeDtypeStruct((NC, NPAD, DH), jnp.float32)]
    scratch = [
        pltpu.VMEM((CHUNKS_PER_TILE, K), jnp.int32),
        pltpu.VMEM((CHUNKS_PER_TILE, K), jnp.int32),
    ]
    scratch += [pltpu.VMEM((K, DH), jnp.float32) for _ in range(RING)]
    scratch += [pltpu.VMEM_SHARED((NPAD, DH), jnp.float32)]
    scratch += [pltpu.SemaphoreType.DMA] * (2 * RING)
    if with_cnt:
        out_type.append(jax.ShapeDtypeStruct((NC, NPAD, CW), jnp.float32))
        scratch += [
            pltpu.VMEM((K, CW), jnp.float32),
            pltpu.VMEM_SHARED((NPAD, CW), jnp.float32),
        ]

    def body(*refs):
        if with_cnt:
            (x2_hbm, src_hbm, dst_hbm, zrow_hbm, zcnt_hbm, ocnt_hbm,
             agg_out, cnt_out, src_v, dst_v, *rest) = refs
            rows = rest[:RING]
            agg_sh = rest[RING]
            gsems = rest[RING + 1:2 * RING + 1]
            ssems = rest[2 * RING + 1:3 * RING + 1]
            ones_v, cnt_sh = rest[3 * RING + 1:]
        else:
            (x2_hbm, src_hbm, dst_hbm, zrow_hbm,
             agg_out, src_v, dst_v, *rest) = refs
            rows = rest[:RING]
            agg_sh = rest[RING]
            gsems = rest[RING + 1:2 * RING + 1]
            ssems = rest[2 * RING + 1:3 * RING + 1]
        cid = lax.axis_index("c")
        sid = lax.axis_index("s")
        row0 = sid * ROWS_PER_TILE
        table = x2_hbm.at[cid]
        pltpu.sync_copy(src_hbm.at[pl.ds(sid * CHUNKS_PER_TILE,
                                         CHUNKS_PER_TILE)], src_v)
        pltpu.sync_copy(dst_hbm.at[pl.ds(sid * CHUNKS_PER_TILE,
                                         CHUNKS_PER_TILE)], dst_v)
        pltpu.sync_copy(zrow_hbm, agg_sh.at[pl.ds(row0, ROWS_PER_TILE)])
        if with_cnt:
            pltpu.sync_copy(zcnt_hbm,
                            cnt_sh.at[pl.ds(row0, ROWS_PER_TILE)])
            pltpu.sync_copy(ocnt_hbm, ones_v)
        plsc.subcore_barrier()

        half = CHUNKS_PER_TILE // 2

        def step(j, b, do_gather, do_swait):
            pltpu.make_async_copy(table.at[src_v.at[j]],
                                  rows[b], gsems[b]).wait()
            pltpu.async_copy(rows[b], agg_sh.at[dst_v.at[j]], ssems[b],
                             add=True)
            if with_cnt:
                @pl.when(jnp.logical_xor(j >= half, cid == 0))
                def _():
                    pltpu.sync_copy(ones_v, cnt_sh.at[dst_v.at[j]],
                                    add=True)
            if do_gather:
                b2 = (b + RING - 1) % RING
                if do_swait:
                    pltpu.make_async_copy(rows[b2],
                                          agg_sh.at[dst_v.at[0]],
                                          ssems[b2]).wait()
                pltpu.async_copy(table.at[src_v.at[j + RING - 1]],
                                 rows[b2], gsems[b2])

        for b in range(RING - 1):
            pltpu.async_copy(table.at[src_v.at[b]], rows[b], gsems[b])
        step(0, 0, do_gather=True, do_swait=False)

        n_mid = CHUNKS_PER_TILE - RING
        assert n_mid % RING == 0

        def ring_iter(jr, carry):
            for br in range(RING):
                step(1 + jr * RING + br, (1 + br) % RING,
                     do_gather=True, do_swait=True)
            return carry

        lax.fori_loop(0, n_mid // RING, ring_iter, 0)
        for j in range(CHUNKS_PER_TILE - RING + 1, CHUNKS_PER_TILE):
            step(j, j % RING, do_gather=False, do_swait=False)
        for b in range(RING):
            pltpu.make_async_copy(rows[b], agg_sh.at[dst_v.at[0]],
                                  ssems[b]).wait()
        plsc.subcore_barrier()
        pltpu.sync_copy(agg_sh.at[pl.ds(row0, ROWS_PER_TILE)],
                        agg_out.at[cid, pl.ds(row0, ROWS_PER_TILE)])
        if with_cnt:
            pltpu.sync_copy(cnt_sh.at[pl.ds(row0, ROWS_PER_TILE)],
                            cnt_out.at[cid, pl.ds(row0, ROWS_PER_TILE)])

    mesh = plsc.VectorSubcoreMesh(core_axis_name="c", subcore_axis_name="s")
    return pl.kernel(body, out_type=tuple(out_type), mesh=mesh,
                     scratch_types=scratch,
                     compiler_params=pltpu.CompilerParams(
                         use_tc_tiling_on_sc=False))


def _tc_layer(aggp, cntp, x2, wl_t, wr_t, b, relu, split_out):
    def body(aggp_ref, cntp_ref, x_ref, wl_ref, wr_ref, b_ref, o_ref):
        cnt = cntp_ref[0, :, 0:1] + cntp_ref[1, :, 0:1]
        inv = 1.0 / jnp.maximum(cnt, 1.0)
        acc = (
            jnp.dot(aggp_ref[0] * inv, wl_ref[0:DH],
                    preferred_element_type=jnp.float32)
            + jnp.dot(aggp_ref[1] * inv, wl_ref[DH:D],
                      preferred_element_type=jnp.float32)
            + jnp.dot(x_ref[0], wr_ref[0:DH],
                      preferred_element_type=jnp.float32)
            + jnp.dot(x_ref[1], wr_ref[DH:D],
                      preferred_element_type=jnp.float32)
            + b_ref[...]
        )
        if relu:
            acc = jnp.maximum(acc, 0.0)
        if split_out:
            o_ref[0] = acc[:, 0:DH]
            o_ref[1] = acc[:, DH:D]
        else:
            o_ref[...] = acc

    blk = BLK
    if split_out:
        out_shape = jax.ShapeDtypeStruct((NC, NPAD, DH), jnp.float32)
        out_spec = pl.BlockSpec((NC, blk, DH), lambda i: (0, i, 0))
    else:
        out_shape = jax.ShapeDtypeStruct((NPAD, D), jnp.float32)
        out_spec = pl.BlockSpec((blk, D), lambda i: (i, 0))

    return pl.pallas_call(
        body,
        grid=(NPAD // blk,),
        in_specs=[
            pl.BlockSpec((NC, blk, DH), lambda i: (0, i, 0)),
            pl.BlockSpec((NC, blk, CW), lambda i: (0, i, 0)),
            pl.BlockSpec((NC, blk, DH), lambda i: (0, i, 0)),
            pl.BlockSpec((D, D), lambda i: (0, 0)),
            pl.BlockSpec((D, D), lambda i: (0, 0)),
            pl.BlockSpec((1, D), lambda i: (0, 0)),
        ],
        out_specs=out_spec,
        out_shape=out_shape,
    )(aggp, cntp, x2, wl_t, wr_t, b)


def kernel(x, edge_index, W1_l, b1_l, W1_r, W2_l, b2_l, W2_r):
    src = edge_index[0].astype(jnp.int32)
    dst = edge_index[1].astype(jnp.int32)
    pad_e = E_PAD - N_EDGES
    src_pad = jnp.concatenate(
        [src, jnp.full((pad_e,), N_NODES, jnp.int32)]).reshape(E_PAD // K, K)
    dst_pad = jnp.concatenate(
        [dst, jnp.full((pad_e,), N_NODES, jnp.int32)]).reshape(E_PAD // K, K)
    x_pad = jnp.concatenate(
        [x, jnp.zeros((NPAD - N_NODES, D), jnp.float32)], axis=0)
    x2 = x_pad.reshape(NPAD, NC, DH).transpose(1, 0, 2)

    zrow = jnp.zeros((ROWS_PER_TILE, DH), jnp.float32)
    zcnt = jnp.zeros((ROWS_PER_TILE, CW), jnp.float32)
    ocnt = jnp.ones((K, CW), jnp.float32)

    sc_agg_cnt = _make_sc_agg(with_cnt=True)
    sc_agg = _make_sc_agg(with_cnt=False)

    aggp1, cntp = sc_agg_cnt(x2, src_pad, dst_pad, zrow, zcnt, ocnt)
    h2 = _tc_layer(aggp1, cntp, x2, W1_l.T, W1_r.T, b1_l.reshape(1, D),
                   relu=True, split_out=True)
    (aggp2,) = sc_agg(h2, src_pad, dst_pad, zrow)
    out_pad = _tc_layer(aggp2, cntp, h2, W2_l.T, W2_r.T, b2_l.reshape(1, D),
                        relu=False, split_out=False)
    return out_pad[:N_NODES]


# speedup vs baseline: 1.0448x; 1.0448x over previous
import jax
import jax.numpy as jnp
from jax import lax
from jax.experimental import pallas as pl
from jax.experimental.pallas import tpu as pltpu
from jax.experimental.pallas import tpu_sc as plsc

N_NODES = 10000
N_EDGES = 320000
D = 128
DH = D // 2

NC = 2
NS = 16
NPAD = 10112
ROWS_PER_TILE = NPAD // NS
K = 128
E_PAD = 327680
EDGES_PER_TILE = E_PAD // NS
CHUNKS_PER_TILE = EDGES_PER_TILE // K
CW = 16
RING = 4

_CHUNK_SIZES = (128, 128, 128, 128, 120)

BLK = NPAD // 4


def _make_sc_agg(with_cnt: bool):
    out_type = [jax.ShapeDtypeStruct((NC, NPAD, DH), jnp.float32)]
    scratch = [
        pltpu.VMEM((CHUNKS_PER_TILE, K), jnp.int32),
        pltpu.VMEM((CHUNKS_PER_TILE, K), jnp.int32),
    ]
    scratch += [pltpu.VMEM((K, DH), jnp.float32) for _ in range(RING)]
    scratch += [pltpu.VMEM_SHARED((NPAD, DH), jnp.float32)]
    scratch += [pltpu.SemaphoreType.DMA] * (2 * RING)
    if with_cnt:
        out_type.append(jax.ShapeDtypeStruct((NC, NPAD, CW), jnp.float32))
        scratch += [
            pltpu.VMEM((K, CW), jnp.float32),
            pltpu.VMEM_SHARED((NPAD, CW), jnp.float32),
        ]

    def body(*refs):
        if with_cnt:
            (x2_hbm, src_hbm, dst_hbm, zrow_hbm, zcnt_hbm, ocnt_hbm,
             agg_out, cnt_out, src_v, dst_v, *rest) = refs
            rows = rest[:RING]
            agg_sh = rest[RING]
            gsems = rest[RING + 1:2 * RING + 1]
            ssems = rest[2 * RING + 1:3 * RING + 1]
            ones_v, cnt_sh = rest[3 * RING + 1:]
        else:
            (x2_hbm, src_hbm, dst_hbm, zrow_hbm,
             agg_out, src_v, dst_v, *rest) = refs
            rows = rest[:RING]
            agg_sh = rest[RING]
            gsems = rest[RING + 1:2 * RING + 1]
            ssems = rest[2 * RING + 1:3 * RING + 1]
        cid = lax.axis_index("c")
        sid = lax.axis_index("s")
        row0 = sid * ROWS_PER_TILE
        table = x2_hbm.at[cid]
        pltpu.sync_copy(src_hbm.at[pl.ds(sid * CHUNKS_PER_TILE,
                                         CHUNKS_PER_TILE)], src_v)
        pltpu.sync_copy(dst_hbm.at[pl.ds(sid * CHUNKS_PER_TILE,
                                         CHUNKS_PER_TILE)], dst_v)
        pltpu.sync_copy(zrow_hbm, rows[0])
        off = 0
        for sz in _CHUNK_SIZES:
            pltpu.sync_copy(rows[0].at[pl.ds(0, sz)],
                            agg_sh.at[pl.ds(row0 + off, sz)])
            off += sz
        if with_cnt:
            pltpu.sync_copy(zcnt_hbm, ones_v)
            o = 0
            for sz in _CHUNK_SIZES:
                pltpu.sync_copy(ones_v.at[pl.ds(0, sz)],
                                cnt_sh.at[pl.ds(row0 + o, sz)])
                o += sz
            pltpu.sync_copy(ocnt_hbm, ones_v)
        plsc.subcore_barrier()

        half = CHUNKS_PER_TILE // 2

        def step(j, b, do_gather, do_swait):
            pltpu.make_async_copy(table.at[src_v.at[j]],
                                  rows[b], gsems[b]).wait()
            pltpu.async_copy(rows[b], agg_sh.at[dst_v.at[j]], ssems[b],
                             add=True)
            if with_cnt:
                @pl.when(jnp.logical_xor(j >= half, cid == 0))
                def _():
                    pltpu.sync_copy(ones_v, cnt_sh.at[dst_v.at[j]],
                                    add=True)
            if do_gather:
                b2 = (b + RING - 1) % RING
                if do_swait:
                    pltpu.make_async_copy(rows[b2],
                                          agg_sh.at[dst_v.at[0]],
                                          ssems[b2]).wait()
                pltpu.async_copy(table.at[src_v.at[j + RING - 1]],
                                 rows[b2], gsems[b2])

        for b in range(RING - 1):
            pltpu.async_copy(table.at[src_v.at[b]], rows[b], gsems[b])
        step(0, 0, do_gather=True, do_swait=False)

        n_mid = CHUNKS_PER_TILE - RING
        assert n_mid % RING == 0

        def ring_iter(jr, carry):
            for br in range(RING):
                step(1 + jr * RING + br, (1 + br) % RING,
                     do_gather=True, do_swait=True)
            return carry

        lax.fori_loop(0, n_mid // RING, ring_iter, 0)
        for j in range(CHUNKS_PER_TILE - RING + 1, CHUNKS_PER_TILE):
            step(j, j % RING, do_gather=False, do_swait=False)
        for b in range(RING):
            pltpu.make_async_copy(rows[b], agg_sh.at[dst_v.at[0]],
                                  ssems[b]).wait()
        plsc.subcore_barrier()
        off = 0
        for sz in _CHUNK_SIZES:
            pltpu.sync_copy(agg_sh.at[pl.ds(row0 + off, sz)],
                            rows[0].at[pl.ds(0, sz)])
            pltpu.sync_copy(rows[0].at[pl.ds(0, sz)],
                            agg_out.at[cid, pl.ds(row0 + off, sz)])
            off += sz
        if with_cnt:
            o = 0
            for sz in _CHUNK_SIZES:
                pltpu.sync_copy(cnt_sh.at[pl.ds(row0 + o, sz)],
                                ones_v.at[pl.ds(0, sz)])
                pltpu.sync_copy(ones_v.at[pl.ds(0, sz)],
                                cnt_out.at[cid, pl.ds(row0 + o, sz)])
                o += sz

    mesh = plsc.VectorSubcoreMesh(core_axis_name="c", subcore_axis_name="s")
    return pl.kernel(body, out_type=tuple(out_type), mesh=mesh,
                     scratch_types=scratch,
                     compiler_params=pltpu.CompilerParams(
                         use_tc_tiling_on_sc=False))


def _tc_layer(aggp, cntp, x2, wl_t, wr_t, b, relu, split_out):
    def body(aggp_ref, cntp_ref, x_ref, wl_ref, wr_ref, b_ref, o_ref):
        cnt = cntp_ref[0, :, 0:1] + cntp_ref[1, :, 0:1]
        inv = 1.0 / jnp.maximum(cnt, 1.0)
        acc = (
            jnp.dot(aggp_ref[0] * inv, wl_ref[0:DH],
                    preferred_element_type=jnp.float32)
            + jnp.dot(aggp_ref[1] * inv, wl_ref[DH:D],
                      preferred_element_type=jnp.float32)
            + jnp.dot(x_ref[0], wr_ref[0:DH],
                      preferred_element_type=jnp.float32)
            + jnp.dot(x_ref[1], wr_ref[DH:D],
                      preferred_element_type=jnp.float32)
            + b_ref[...]
        )
        if relu:
            acc = jnp.maximum(acc, 0.0)
        if split_out:
            o_ref[0] = acc[:, 0:DH]
            o_ref[1] = acc[:, DH:D]
        else:
            o_ref[...] = acc

    blk = BLK
    if split_out:
        out_shape = jax.ShapeDtypeStruct((NC, NPAD, DH), jnp.float32)
        out_spec = pl.BlockSpec((NC, blk, DH), lambda i: (0, i, 0))
    else:
        out_shape = jax.ShapeDtypeStruct((NPAD, D), jnp.float32)
        out_spec = pl.BlockSpec((blk, D), lambda i: (i, 0))

    return pl.pallas_call(
        body,
        grid=(NPAD // blk,),
        in_specs=[
            pl.BlockSpec((NC, blk, DH), lambda i: (0, i, 0)),
            pl.BlockSpec((NC, blk, CW), lambda i: (0, i, 0)),
            pl.BlockSpec((NC, blk, DH), lambda i: (0, i, 0)),
            pl.BlockSpec((D, D), lambda i: (0, 0)),
            pl.BlockSpec((D, D), lambda i: (0, 0)),
            pl.BlockSpec((1, D), lambda i: (0, 0)),
        ],
        out_specs=out_spec,
        out_shape=out_shape,
    )(aggp, cntp, x2, wl_t, wr_t, b)


def kernel(x, edge_index, W1_l, b1_l, W1_r, W2_l, b2_l, W2_r):
    src = edge_index[0].astype(jnp.int32)
    dst = edge_index[1].astype(jnp.int32)
    pad_e = E_PAD - N_EDGES
    src_pad = jnp.concatenate(
        [src, jnp.full((pad_e,), N_NODES, jnp.int32)]).reshape(E_PAD // K, K)
    dst_pad = jnp.concatenate(
        [dst, jnp.full((pad_e,), N_NODES, jnp.int32)]).reshape(E_PAD // K, K)
    x_pad = jnp.concatenate(
        [x, jnp.zeros((NPAD - N_NODES, D), jnp.float32)], axis=0)
    x2 = x_pad.reshape(NPAD, NC, DH).transpose(1, 0, 2)

    zrow = jnp.zeros((K, DH), jnp.float32)
    zcnt = jnp.zeros((K, CW), jnp.float32)
    ocnt = jnp.ones((K, CW), jnp.float32)

    sc_agg_cnt = _make_sc_agg(with_cnt=True)
    sc_agg = _make_sc_agg(with_cnt=False)

    aggp1, cntp = sc_agg_cnt(x2, src_pad, dst_pad, zrow, zcnt, ocnt)
    h2 = _tc_layer(aggp1, cntp, x2, W1_l.T, W1_r.T, b1_l.reshape(1, D),
                   relu=True, split_out=True)
    (aggp2,) = sc_agg(h2, src_pad, dst_pad, zrow)
    out_pad = _tc_layer(aggp2, cntp, h2, W2_l.T, W2_r.T, b2_l.reshape(1, D),
                        relu=False, split_out=False)
    return out_pad[:N_NODES]


# speedup vs baseline: 1.0465x; 1.0017x over previous
import jax
import jax.numpy as jnp
from jax import lax
from jax.experimental import pallas as pl
from jax.experimental.pallas import tpu as pltpu
from jax.experimental.pallas import tpu_sc as plsc

N_NODES = 10000
N_EDGES = 320000
D = 128
DH = D // 2

NC = 2
NS = 16
NPAD = 10112
ROWS_PER_TILE = NPAD // NS
K = 128
E_PAD = 327680
EDGES_PER_TILE = E_PAD // NS
CHUNKS_PER_TILE = EDGES_PER_TILE // K
CW = 16
RING = 4

_CHUNK_SIZES = (128, 128, 128, 128, 120)

BLK = NPAD // 4


def _make_sc_agg(with_cnt: bool):
    out_type = [jax.ShapeDtypeStruct((NC, NPAD, DH), jnp.float32)]
    scratch = [
        pltpu.VMEM((CHUNKS_PER_TILE, K), jnp.int32),
        pltpu.VMEM((CHUNKS_PER_TILE, K), jnp.int32),
    ]
    scratch += [pltpu.VMEM((K, DH), jnp.float32) for _ in range(RING)]
    scratch += [pltpu.VMEM_SHARED((NPAD, DH), jnp.float32)]
    scratch += [pltpu.SemaphoreType.DMA] * (RING + 1)
    if with_cnt:
        out_type.append(jax.ShapeDtypeStruct((NC, NPAD, CW), jnp.float32))
        scratch += [
            pltpu.VMEM((K, CW), jnp.float32),
            pltpu.VMEM_SHARED((NPAD, CW), jnp.float32),
        ]

    def body(*refs):
        if with_cnt:
            (x2_hbm, src_hbm, dst_hbm, zrow_hbm, zcnt_hbm, ocnt_hbm,
             agg_out, cnt_out, src_v, dst_v, *rest) = refs
            rows = rest[:RING]
            agg_sh = rest[RING]
            gsems = rest[RING + 1:2 * RING + 1]
            ssem = rest[2 * RING + 1]
            ones_v, cnt_sh = rest[2 * RING + 2:]
        else:
            (x2_hbm, src_hbm, dst_hbm, zrow_hbm,
             agg_out, src_v, dst_v, *rest) = refs
            rows = rest[:RING]
            agg_sh = rest[RING]
            gsems = rest[RING + 1:2 * RING + 1]
            ssem = rest[2 * RING + 1]
        cid = lax.axis_index("c")
        sid = lax.axis_index("s")
        row0 = sid * ROWS_PER_TILE
        table = x2_hbm.at[cid]
        pltpu.sync_copy(src_hbm.at[pl.ds(sid * CHUNKS_PER_TILE,
                                         CHUNKS_PER_TILE)], src_v)
        pltpu.sync_copy(dst_hbm.at[pl.ds(sid * CHUNKS_PER_TILE,
                                         CHUNKS_PER_TILE)], dst_v)
        pltpu.sync_copy(zrow_hbm, rows[0])
        off = 0
        for sz in _CHUNK_SIZES:
            pltpu.sync_copy(rows[0].at[pl.ds(0, sz)],
                            agg_sh.at[pl.ds(row0 + off, sz)])
            off += sz
        if with_cnt:
            pltpu.sync_copy(zcnt_hbm, ones_v)
            o = 0
            for sz in _CHUNK_SIZES:
                pltpu.sync_copy(ones_v.at[pl.ds(0, sz)],
                                cnt_sh.at[pl.ds(row0 + o, sz)])
                o += sz
            pltpu.sync_copy(ocnt_hbm, ones_v)
        plsc.subcore_barrier()

        half = CHUNKS_PER_TILE // 2

        def step(j, b, do_gather, do_swait):
            pltpu.make_async_copy(table.at[src_v.at[j]],
                                  rows[b], gsems[b]).wait()
            if do_swait:
                pltpu.make_async_copy(rows[b], agg_sh.at[dst_v.at[0]],
                                      ssem).wait()
            if with_cnt:
                @pl.when(jnp.logical_xor(j >= half, cid == 0))
                def _():
                    pltpu.sync_copy(ones_v, cnt_sh.at[dst_v.at[j]],
                                    add=True)
            pltpu.async_copy(rows[b], agg_sh.at[dst_v.at[j]], ssem,
                             add=True)
            if do_gather:
                b2 = (b + RING - 1) % RING
                pltpu.async_copy(table.at[src_v.at[j + RING - 1]],
                                 rows[b2], gsems[b2])

        for b in range(RING - 1):
            pltpu.async_copy(table.at[src_v.at[b]], rows[b], gsems[b])
        step(0, 0, do_gather=True, do_swait=False)

        n_mid = CHUNKS_PER_TILE - RING
        assert n_mid % RING == 0

        def ring_iter(jr, carry):
            for br in range(RING):
                step(1 + jr * RING + br, (1 + br) % RING,
                     do_gather=True, do_swait=True)
            return carry

        lax.fori_loop(0, n_mid // RING, ring_iter, 0)
        for j in range(CHUNKS_PER_TILE - RING + 1, CHUNKS_PER_TILE):
            step(j, j % RING, do_gather=False, do_swait=True)
        pltpu.make_async_copy(rows[0], agg_sh.at[dst_v.at[0]],
                              ssem).wait()
        plsc.subcore_barrier()
        off = 0
        for sz in _CHUNK_SIZES:
            pltpu.sync_copy(agg_sh.at[pl.ds(row0 + off, sz)],
                            rows[0].at[pl.ds(0, sz)])
            pltpu.sync_copy(rows[0].at[pl.ds(0, sz)],
                            agg_out.at[cid, pl.ds(row0 + off, sz)])
            off += sz
        if with_cnt:
            o = 0
            for sz in _CHUNK_SIZES:
                pltpu.sync_copy(cnt_sh.at[pl.ds(row0 + o, sz)],
                                ones_v.at[pl.ds(0, sz)])
                pltpu.sync_copy(ones_v.at[pl.ds(0, sz)],
                                cnt_out.at[cid, pl.ds(row0 + o, sz)])
                o += sz

    mesh = plsc.VectorSubcoreMesh(core_axis_name="c", subcore_axis_name="s")
    return pl.kernel(body, out_type=tuple(out_type), mesh=mesh,
                     scratch_types=scratch,
                     compiler_params=pltpu.CompilerParams(
                         use_tc_tiling_on_sc=False))


def _tc_layer(aggp, cntp, x2, wl_t, wr_t, b, relu, split_out):
    def body(aggp_ref, cntp_ref, x_ref, wl_ref, wr_ref, b_ref, o_ref):
        cnt = cntp_ref[0, :, 0:1] + cntp_ref[1, :, 0:1]
        inv = 1.0 / jnp.maximum(cnt, 1.0)
        acc = (
            jnp.dot(aggp_ref[0] * inv, wl_ref[0:DH],
                    preferred_element_type=jnp.float32)
            + jnp.dot(aggp_ref[1] * inv, wl_ref[DH:D],
                      preferred_element_type=jnp.float32)
            + jnp.dot(x_ref[0], wr_ref[0:DH],
                      preferred_element_type=jnp.float32)
            + jnp.dot(x_ref[1], wr_ref[DH:D],
                      preferred_element_type=jnp.float32)
            + b_ref[...]
        )
        if relu:
            acc = jnp.maximum(acc, 0.0)
        if split_out:
            o_ref[0] = acc[:, 0:DH]
            o_ref[1] = acc[:, DH:D]
        else:
            o_ref[...] = acc

    blk = BLK
    if split_out:
        out_shape = jax.ShapeDtypeStruct((NC, NPAD, DH), jnp.float32)
        out_spec = pl.BlockSpec((NC, blk, DH), lambda i: (0, i, 0))
    else:
        out_shape = jax.ShapeDtypeStruct((NPAD, D), jnp.float32)
        out_spec = pl.BlockSpec((blk, D), lambda i: (i, 0))

    return pl.pallas_call(
        body,
        grid=(NPAD // blk,),
        in_specs=[
            pl.BlockSpec((NC, blk, DH), lambda i: (0, i, 0)),
            pl.BlockSpec((NC, blk, CW), lambda i: (0, i, 0)),
            pl.BlockSpec((NC, blk, DH), lambda i: (0, i, 0)),
            pl.BlockSpec((D, D), lambda i: (0, 0)),
            pl.BlockSpec((D, D), lambda i: (0, 0)),
            pl.BlockSpec((1, D), lambda i: (0, 0)),
        ],
        out_specs=out_spec,
        out_shape=out_shape,
    )(aggp, cntp, x2, wl_t, wr_t, b)


def kernel(x, edge_index, W1_l, b1_l, W1_r, W2_l, b2_l, W2_r):
    src = edge_index[0].astype(jnp.int32)
    dst = edge_index[1].astype(jnp.int32)
    pad_e = E_PAD - N_EDGES
    src_pad = jnp.concatenate(
        [src, jnp.full((pad_e,), N_NODES, jnp.int32)]).reshape(E_PAD // K, K)
    dst_pad = jnp.concatenate(
        [dst, jnp.full((pad_e,), N_NODES, jnp.int32)]).reshape(E_PAD // K, K)
    x_pad = jnp.concatenate(
        [x, jnp.zeros((NPAD - N_NODES, D), jnp.float32)], axis=0)
    x2 = x_pad.reshape(NPAD, NC, DH).transpose(1, 0, 2)

    zrow = jnp.zeros((K, DH), jnp.float32)
    zcnt = jnp.zeros((K, CW), jnp.float32)
    ocnt = jnp.ones((K, CW), jnp.float32)

    sc_agg_cnt = _make_sc_agg(with_cnt=True)
    sc_agg = _make_sc_agg(with_cnt=False)

    aggp1, cntp = sc_agg_cnt(x2, src_pad, dst_pad, zrow, zcnt, ocnt)
    h2 = _tc_layer(aggp1, cntp, x2, W1_l.T, W1_r.T, b1_l.reshape(1, D),
                   relu=True, split_out=True)
    (aggp2,) = sc_agg(h2, src_pad, dst_pad, zrow)
    out_pad = _tc_layer(aggp2, cntp, h2, W2_l.T, W2_r.T, b2_l.reshape(1, D),
                        relu=False, split_out=False)
    return out_pad[:N_NODES]
